# baseline (speedup 1.0000x reference)
import functools

import jax
import jax.numpy as jnp
from jax import lax
from jax.experimental import pallas as pl
from jax.experimental.pallas import tpu as pltpu
from jax.experimental.pallas import tpu_sc as plsc

_N_CLASSES = 100000
_B = 2048
_SMOOTHING = 0.1
_BASE = _SMOOTHING / (_N_CLASSES - 1)
_CONF = 1.0 - _SMOOTHING

_SC_ROWS = 10240
_SPLIT = _N_CLASSES - _SC_ROWS
_BR = 2992
_NBLK = _SPLIT // _BR

_NW = 32
_RPW = _SC_ROWS // _NW
_NSLAB = _RPW // 16
_CPW = _B // _NW


def _tc_body(t_ref, x_ref, sum_ref, gsum_ref):
    j = pl.program_id(0)

    @pl.when(j == 0)
    def _init():
        sum_ref[...] = jnp.zeros((1, 1), jnp.float32)
        gsum_ref[...] = jnp.zeros((1, 1), jnp.float32)

    x = x_ref[...]
    rows = j * _BR + lax.broadcasted_iota(jnp.int32, (_BR, 1), 0)
    hit = rows == t_ref[...]
    sum_ref[...] += jnp.sum(x).reshape(1, 1)
    gsum_ref[...] += jnp.sum(jnp.where(hit, x, 0.0)).reshape(1, 1)


def _tc_part(t2d, xT):
    return pl.pallas_call(
        _tc_body,
        grid=(_NBLK,),
        in_specs=[
            pl.BlockSpec((1, _B), lambda j: (0, 0)),
            pl.BlockSpec((_BR, _B), lambda j: (j, 0)),
        ],
        out_specs=[
            pl.BlockSpec((1, 1), lambda j: (0, 0)),
            pl.BlockSpec((1, 1), lambda j: (0, 0)),
        ],
        out_shape=[
            jax.ShapeDtypeStruct((1, 1), jnp.float32),
            jax.ShapeDtypeStruct((1, 1), jnp.float32),
        ],
        compiler_params=pltpu.CompilerParams(
            vmem_limit_bytes=64 * 1024 * 1024),
    )(t2d, xT)


@functools.partial(
    pl.kernel,
    mesh=plsc.VectorSubcoreMesh(core_axis_name="c", subcore_axis_name="s"),
    out_type=[
        jax.ShapeDtypeStruct((_NW, 16), jnp.float32),
        jax.ShapeDtypeStruct((_NW, 16), jnp.float32),
    ],
    scratch_types=[
        pltpu.VMEM((_CPW,), jnp.int32),
        pltpu.VMEM((16,), jnp.float32),
        pltpu.VMEM((16, _B), jnp.float32),
        pltpu.VMEM((16, _B), jnp.float32),
        pltpu.VMEM((16,), jnp.float32),
        pltpu.VMEM((16,), jnp.float32),
        pltpu.SemaphoreType.DMA,
        pltpu.SemaphoreType.DMA,
    ],
    compiler_params=pltpu.CompilerParams(use_tc_tiling_on_sc=True),
)
def _sc_part(tgt_hbm, xt_hbm, dsum_out, gsum_out,
             tgt_v, row_v, buf0, buf1, dacc_v, gacc_v, sem0, sem1):
    wid = lax.axis_index("s") * 2 + lax.axis_index("c")

    col0 = wid * _CPW
    pltpu.sync_copy(tgt_hbm.at[pl.ds(col0, _CPW)], tgt_v)
    iota = lax.iota(jnp.int32, 16)
    gacc = jnp.zeros((16,), jnp.float32)
    for j in range(_CPW // 16):
        t_vec = tgt_v[pl.ds(j * 16, 16)]
        for i in range(16):
            t = t_vec[i]
            in_sc = t >= _SPLIT
            trow = jnp.where(in_sc, t, _SPLIT)
            b = col0 + j * 16 + i
            start = (b // 16) * 16
            pltpu.sync_copy(xt_hbm.at[trow, pl.ds(start, 16)], row_v)
            sel = jnp.where(in_sc, b - start, -1)
            gacc = gacc + jnp.where(iota == sel, row_v[...], 0.0)
    gacc_v[...] = gacc
    pltpu.sync_copy(gacc_v, gsum_out.at[wid])

    row0 = _SPLIT + wid * _RPW
    bufs = (buf0, buf1)
    sems = (sem0, sem1)

    def _slab_slice(q):
        return xt_hbm.at[pl.ds(row0 + q * 16, 16), pl.ds(0, _B)]

    pltpu.async_copy(_slab_slice(0), buf0, sem0)
    pltpu.async_copy(_slab_slice(1), buf1, sem1)

    def _reduce_slab(buf, acc):
        def inner(i, a):
            for r in range(16):
                a = a + buf[r, pl.ds(i * 16, 16)]
            return a
        return lax.fori_loop(0, _B // 16, inner, acc)

    def pair_body(k, dacc):
        for par in range(2):
            q = 2 * k + par
            pltpu.make_async_copy(_slab_slice(q), bufs[par], sems[par]).wait()
            dacc = _reduce_slab(bufs[par], dacc)

            @pl.when(q + 2 < _NSLAB)
            def _issue():
                pltpu.async_copy(_slab_slice(q + 2), bufs[par], sems[par])
        return dacc

    dacc = lax.fori_loop(0, _NSLAB // 2, pair_body,
                         jnp.zeros((16,), jnp.float32))
    dacc_v[...] = dacc
    pltpu.sync_copy(dacc_v, dsum_out.at[wid])


def kernel(lsm, target):
    tgt = target.astype(jnp.int32)
    t2d = tgt.reshape(1, _B)
    xT = lsm.T
    tc_total, tc_gsum = _tc_part(t2d, xT)
    sc_dsum, sc_gsum = _sc_part(tgt, xT)
    total = tc_total[0, 0] + jnp.sum(sc_dsum)
    gsum = tc_gsum[0, 0] + jnp.sum(sc_gsum)
    scale = jnp.float32(_CONF - _BASE)
    return -(jnp.float32(_BASE) * total + scale * gsum) / jnp.float32(_B)


# speedup vs baseline: 1.0230x; 1.0230x over previous
import functools

import jax
import jax.numpy as jnp
from jax import lax
from jax.experimental import pallas as pl
from jax.experimental.pallas import tpu as pltpu
from jax.experimental.pallas import tpu_sc as plsc

_N_CLASSES = 100000
_B = 2048
_SMOOTHING = 0.1
_BASE = _SMOOTHING / (_N_CLASSES - 1)
_CONF = 1.0 - _SMOOTHING

_SC_ROWS = 30720
_SPLIT = _N_CLASSES - _SC_ROWS
_BR = 3464
_NBLK = _SPLIT // _BR

_NW = 32
_RPW = _SC_ROWS // _NW
_NSLAB = _RPW // 16
_CPW = _B // _NW


def _tc_body(t_ref, x_ref, sum_ref, gsum_ref):
    j = pl.program_id(0)

    @pl.when(j == 0)
    def _init():
        sum_ref[...] = jnp.zeros((1, 1), jnp.float32)
        gsum_ref[...] = jnp.zeros((1, 1), jnp.float32)

    x = x_ref[...]
    rows = j * _BR + lax.broadcasted_iota(jnp.int32, (_BR, 1), 0)
    hit = rows == t_ref[...]
    sum_ref[...] += jnp.sum(x).reshape(1, 1)
    gsum_ref[...] += jnp.sum(jnp.where(hit, x, 0.0)).reshape(1, 1)


def _tc_part(t2d, xT):
    return pl.pallas_call(
        _tc_body,
        grid=(_NBLK,),
        in_specs=[
            pl.BlockSpec((1, _B), lambda j: (0, 0)),
            pl.BlockSpec((_BR, _B), lambda j: (j, 0)),
        ],
        out_specs=[
            pl.BlockSpec((1, 1), lambda j: (0, 0)),
            pl.BlockSpec((1, 1), lambda j: (0, 0)),
        ],
        out_shape=[
            jax.ShapeDtypeStruct((1, 1), jnp.float32),
            jax.ShapeDtypeStruct((1, 1), jnp.float32),
        ],
        compiler_params=pltpu.CompilerParams(
            vmem_limit_bytes=64 * 1024 * 1024),
    )(t2d, xT)


@functools.partial(
    pl.kernel,
    mesh=plsc.VectorSubcoreMesh(core_axis_name="c", subcore_axis_name="s"),
    out_type=[
        jax.ShapeDtypeStruct((_NW, 16), jnp.float32),
        jax.ShapeDtypeStruct((_NW, 16), jnp.float32),
    ],
    scratch_types=[
        pltpu.VMEM((_CPW,), jnp.int32),
        pltpu.VMEM((16,), jnp.float32),
        pltpu.VMEM((16, _B), jnp.float32),
        pltpu.VMEM((16, _B), jnp.float32),
        pltpu.VMEM((16,), jnp.float32),
        pltpu.VMEM((16,), jnp.float32),
        pltpu.SemaphoreType.DMA,
        pltpu.SemaphoreType.DMA,
    ],
    compiler_params=pltpu.CompilerParams(use_tc_tiling_on_sc=True),
)
def _sc_part(tgt_hbm, xt_hbm, dsum_out, gsum_out,
             tgt_v, row_v, buf0, buf1, dacc_v, gacc_v, sem0, sem1):
    wid = lax.axis_index("s") * 2 + lax.axis_index("c")

    col0 = wid * _CPW
    pltpu.sync_copy(tgt_hbm.at[pl.ds(col0, _CPW)], tgt_v)
    iota = lax.iota(jnp.int32, 16)
    gacc = jnp.zeros((16,), jnp.float32)
    for j in range(_CPW // 16):
        t_vec = tgt_v[pl.ds(j * 16, 16)]
        for i in range(16):
            t = t_vec[i]
            in_sc = t >= _SPLIT
            trow = jnp.where(in_sc, t, _SPLIT)
            b = col0 + j * 16 + i
            start = (b // 16) * 16
            pltpu.sync_copy(xt_hbm.at[trow, pl.ds(start, 16)], row_v)
            sel = jnp.where(in_sc, b - start, -1)
            gacc = gacc + jnp.where(iota == sel, row_v[...], 0.0)
    gacc_v[...] = gacc
    pltpu.sync_copy(gacc_v, gsum_out.at[wid])

    row0 = _SPLIT + wid * _RPW
    bufs = (buf0, buf1)
    sems = (sem0, sem1)

    def _slab_slice(q):
        return xt_hbm.at[pl.ds(row0 + q * 16, 16), pl.ds(0, _B)]

    pltpu.async_copy(_slab_slice(0), buf0, sem0)
    pltpu.async_copy(_slab_slice(1), buf1, sem1)

    def _reduce_slab(buf, acc):
        def inner(i, a):
            for r in range(16):
                a = a + buf[r, pl.ds(i * 16, 16)]
            return a
        return lax.fori_loop(0, _B // 16, inner, acc)

    def pair_body(k, dacc):
        for par in range(2):
            q = 2 * k + par
            pltpu.make_async_copy(_slab_slice(q), bufs[par], sems[par]).wait()
            dacc = _reduce_slab(bufs[par], dacc)

            @pl.when(q + 2 < _NSLAB)
            def _issue():
                pltpu.async_copy(_slab_slice(q + 2), bufs[par], sems[par])
        return dacc

    dacc = lax.fori_loop(0, _NSLAB // 2, pair_body,
                         jnp.zeros((16,), jnp.float32))
    dacc_v[...] = dacc
    pltpu.sync_copy(dacc_v, dsum_out.at[wid])


def kernel(lsm, target):
    tgt = target.astype(jnp.int32)
    t2d = tgt.reshape(1, _B)
    xT = lsm.T
    tc_total, tc_gsum = _tc_part(t2d, xT)
    sc_dsum, sc_gsum = _sc_part(tgt, xT)
    total = tc_total[0, 0] + jnp.sum(sc_dsum)
    gsum = tc_gsum[0, 0] + jnp.sum(sc_gsum)
    scale = jnp.float32(_CONF - _BASE)
    return -(jnp.float32(_BASE) * total + scale * gsum) / jnp.float32(_B)
